# baseline (speedup 1.0000x reference)
import jax
import jax.numpy as jnp
from jax.experimental import pallas as pl

_BLK = 1000
_BIAS = 131072
_FMAX = 262143


def _tower_body(f_ref, w1_ref, w2_ref, w3_ref, b3_ref, off_ref):
    f = f_ref[...]
    h = jnp.maximum(jnp.dot(f, w1_ref[...], preferred_element_type=jnp.float32), 0.0)
    h = jnp.maximum(jnp.dot(h, w2_ref[...], preferred_element_type=jnp.float32), 0.0)
    off_ref[...] = (
        jnp.dot(h, w3_ref[...], preferred_element_type=jnp.float32) + b3_ref[0:1, :]
    )


def _tower(feats_F, W1, W2, W3, b3):
    n = feats_F.shape[0]
    w3p = jnp.zeros((W3.shape[0], 128), jnp.float32).at[:, :3].set(W3)
    b3p = jnp.zeros((8, 128), jnp.float32).at[0, :3].set(b3)
    offp = pl.pallas_call(
        _tower_body,
        grid=(n // _BLK,),
        in_specs=[
            pl.BlockSpec((_BLK, 128), lambda i: (i, 0)),
            pl.BlockSpec((128, 64), lambda i: (0, 0)),
            pl.BlockSpec((64, 32), lambda i: (0, 0)),
            pl.BlockSpec((32, 128), lambda i: (0, 0)),
            pl.BlockSpec((8, 128), lambda i: (0, 0)),
        ],
        out_specs=pl.BlockSpec((_BLK, 128), lambda i: (i, 0)),
        out_shape=jax.ShapeDtypeStruct((n, 128), jnp.float32),
    )(feats_F, W1, W2, w3p, b3p)
    return offp


def kernel(feats_F, feats_C, W1, W2, W3, b3):
    n = feats_F.shape[0]
    offp = _tower(feats_F, W1, W2, W3, b3)
    offsets = offp[:, :3]

    off_int = (jnp.sign(offsets) * jnp.expm1(jnp.abs(offsets))).astype(jnp.int32)
    new_coords = feats_C.at[:, 1:].add(off_int)

    w = new_coords[:, 0]
    xu = jnp.clip(new_coords[:, 1] + _BIAS, 0, _FMAX)
    yu = jnp.clip(new_coords[:, 2] + _BIAS, 0, _FMAX)
    zu = jnp.clip(new_coords[:, 3] + _BIAS, 0, _FMAX)
    hi = (w << 24) | (xu << 6) | (yu >> 12)
    lo = ((yu & 0xFFF) << 18) | zu

    idx = jnp.arange(n, dtype=jnp.int32)
    hi_s, lo_s, idx_s = jax.lax.sort((hi, lo, idx), num_keys=2)

    flag = jnp.concatenate(
        [
            jnp.ones((1,), jnp.int32),
            ((hi_s[1:] != hi_s[:-1]) | (lo_s[1:] != lo_s[:-1])).astype(jnp.int32),
        ]
    )
    uid_s = jnp.cumsum(flag) - 1

    inverse = jnp.zeros((n,), jnp.int32).at[idx_s].set(uid_s, mode="drop")
    out_coords = (
        jnp.zeros((n, 4), jnp.int32).at[uid_s].set(new_coords[idx_s], mode="drop")
    )
    counts = jax.ops.segment_sum(jnp.ones((n,), jnp.int32), uid_s, num_segments=n)
    cpp = counts[inverse][:, None].astype(feats_F.dtype)
    out_feats = jax.ops.segment_sum(feats_F / cpp, inverse, num_segments=n)
    out_scores = jnp.log1p(counts.astype(feats_F.dtype))[:, None]
    return (offsets, out_coords, out_feats, out_scores, inverse.astype(jnp.int64))
